# baseline (speedup 1.0000x reference)
import functools

import jax
import jax.numpy as jnp
from jax import lax
from jax.experimental import pallas as pl
from jax.experimental.pallas import tpu as pltpu
from jax.experimental.pallas import tpu_sc as plsc

T = 2048
D = 1024
H = 2048
SH = 1024
E = 8
K = 2
N = T * K
BLK = 256
BLK_SH = 8
GMAX = 23
PADN = GMAX * BLK
BEXP_N = 32
NC, NS, L = 2, 16, 16
NW = NC * NS
SLOTS_W = N // NW
CH_W = SLOTS_W // L
TOK_W = T // NW


def _sc_mesh():
    return plsc.VectorSubcoreMesh(
        core_axis_name="c", subcore_axis_name="s", num_cores=NC, num_subcores=NS
    )


def _gate_body(x_ref, gw_ref, w_ref, i_ref):
    xb = x_ref[...]
    gw = gw_ref[...]
    logits = lax.dot_general(xb, gw, (((1,), (1,)), ((), ())),
                             preferred_element_type=jnp.float32)
    m = jnp.max(logits, axis=-1, keepdims=True)
    p = jnp.exp(logits - m)
    probs = p / jnp.sum(p, axis=-1, keepdims=True)
    lane = lax.broadcasted_iota(jnp.int32, probs.shape, 1)
    w1 = jnp.max(probs, axis=-1, keepdims=True)
    i1 = jnp.min(jnp.where(probs == w1, lane, E), axis=-1, keepdims=True)
    probs2 = jnp.where(lane == i1, -jnp.inf, probs)
    w2 = jnp.max(probs2, axis=-1, keepdims=True)
    i2 = jnp.min(jnp.where(probs2 == w2, lane, E), axis=-1, keepdims=True)
    w_ref[...] = jnp.concatenate([w1, w2], axis=1)
    i_ref[...] = jnp.concatenate([i1, i2], axis=1)


def _gate(xf, gate_w):
    tb = 256
    return pl.pallas_call(
        _gate_body,
        grid=(T // tb,),
        in_specs=[
            pl.BlockSpec((tb, D), lambda g: (g, 0)),
            pl.BlockSpec((E, D), lambda g: (0, 0)),
        ],
        out_specs=[
            pl.BlockSpec((tb, K), lambda g: (g, 0)),
            pl.BlockSpec((tb, K), lambda g: (g, 0)),
        ],
        out_shape=[
            jax.ShapeDtypeStruct((T, K), jnp.float32),
            jax.ShapeDtypeStruct((T, K), jnp.int32),
        ],
    )(xf, gate_w)


def _route_body(eidx_hbm, x_hbm, pos_hbm, bexp_hbm, xs_hbm,
                idx_v, pos_v, base_v, bexp_v, xrow_a, xrow_b,
                gsem_a, gsem_b, ssem_a, ssem_b):
    wid = lax.axis_index("s") * NC + lax.axis_index("c")
    lane = lax.iota(jnp.int32, L)
    pltpu.sync_copy(eidx_hbm, idx_v)

    def hist_upto(nchunks):
        def body(i, accs):
            v = idx_v[pl.ds(pl.multiple_of(i * L, L), L)]
            return tuple(accs[e] + (v == e).astype(jnp.int32) for e in range(E))
        accs = lax.fori_loop(0, nchunks,
                             body, tuple(jnp.zeros((L,), jnp.int32) for _ in range(E)))
        h = jnp.zeros((L,), jnp.int32)
        for e in range(E):
            h = jnp.where(lane == e, jnp.sum(accs[e]), h)
        return h

    tot = hist_upto(N // L)
    pre = hist_upto(wid * CH_W)

    padded = ((tot + (BLK - 1)) >> BLK_SH) << BLK_SH
    starts = plsc.cumsum(padded) - padded
    base_v[...] = starts + pre

    cvec = jnp.zeros((L,), jnp.int32)
    for c in range(CH_W):
        v = idx_v[pl.ds(pl.multiple_of((wid * CH_W + c) * L, L), L)]
        r = jnp.zeros((L,), jnp.int32)
        for e in range(E):
            m = v == e
            mi = m.astype(jnp.int32)
            cs = plsc.cumsum(mi)
            ce = jnp.sum(jnp.where(lane == e, cvec, 0))
            r = jnp.where(m, cs - 1 + ce, r)
            cvec = jnp.where(lane == e, cvec + jnp.sum(mi), cvec)
        g = plsc.load_gather(base_v, [v])
        pos_v[pl.ds(c * L, L)] = g + r
    pltpu.sync_copy(pos_v, pos_hbm.at[pl.ds(wid * SLOTS_W, SLOTS_W)])

    @pl.when(wid == 0)
    def _():
        lastp = jnp.max(jnp.where(tot > 0, lane, 0))
        for gi in range(BEXP_N // L):
            b256 = (lane + gi * L) * BLK
            acc = jnp.zeros((L,), jnp.int32) + lastp
            for e in range(E):
                s_e = jnp.sum(jnp.where(lane == e, starts, 0))
                p_e = jnp.sum(jnp.where(lane == e, padded, 0))
                acc = jnp.where((b256 >= s_e) & (b256 < s_e + p_e), e, acc)
            bexp_v[pl.ds(gi * L, L)] = acc
        pltpu.sync_copy(bexp_v, bexp_hbm)

    bufs = (xrow_a, xrow_b)
    gsems = (gsem_a, gsem_b)
    ssems = (ssem_a, ssem_b)

    def tokv(c):
        return (wid * SLOTS_W + c * L + lane) >> 1

    gdesc = {}
    sdesc = {}
    for c in range(min(2, CH_W)):
        gdesc[c] = pltpu.async_copy(x_hbm.at[tokv(c)], bufs[c % 2], gsems[c % 2])
    for c in range(CH_W):
        sel = c % 2
        gdesc[c].wait()
        p = pos_v[pl.ds(c * L, L)]
        sdesc[c] = pltpu.async_copy(bufs[sel], xs_hbm.at[p], ssems[sel])
        if c + 2 < CH_W:
            sdesc[c].wait()
            gdesc[c + 2] = pltpu.async_copy(x_hbm.at[tokv(c + 2)], bufs[sel], gsems[sel])
    sdesc[CH_W - 2].wait()
    sdesc[CH_W - 1].wait()


def _route_dispatch(eidx, xf):
    fn = pl.kernel(
        _route_body,
        out_type=(
            jax.ShapeDtypeStruct((N,), jnp.int32),
            jax.ShapeDtypeStruct((BEXP_N,), jnp.int32),
            jax.ShapeDtypeStruct((PADN, D), jnp.float32),
        ),
        mesh=_sc_mesh(),
        compiler_params=pltpu.CompilerParams(needs_layout_passes=False),
        scratch_types=[
            pltpu.VMEM((N,), jnp.int32),
            pltpu.VMEM((SLOTS_W,), jnp.int32),
            pltpu.VMEM((L,), jnp.int32),
            pltpu.VMEM((BEXP_N,), jnp.int32),
            pltpu.VMEM((L, D), jnp.float32),
            pltpu.VMEM((L, D), jnp.float32),
            pltpu.SemaphoreType.DMA,
            pltpu.SemaphoreType.DMA,
            pltpu.SemaphoreType.DMA,
            pltpu.SemaphoreType.DMA,
        ],
    )
    return fn(eidx, xf)


def _swiglu_block(xb, w1b, w3b, w2b):
    u = lax.dot_general(xb, w1b, (((1,), (1,)), ((), ())),
                        preferred_element_type=jnp.float32)
    v = lax.dot_general(xb, w3b, (((1,), (1,)), ((), ())),
                        preferred_element_type=jnp.float32)
    h = u * jax.nn.sigmoid(u) * v
    return lax.dot_general(h, w2b, (((1,), (1,)), ((), ())),
                           preferred_element_type=jnp.float32)


def _gmm_body(be_ref, xs_ref, w1_ref, w3_ref, w2_ref, o_ref):
    o_ref[...] = _swiglu_block(xs_ref[...], w1_ref[0], w3_ref[0], w2_ref[0])


def _group_mm(bexp, xs, w1, w3, w2):
    grid_spec = pltpu.PrefetchScalarGridSpec(
        num_scalar_prefetch=1,
        grid=(GMAX,),
        in_specs=[
            pl.BlockSpec((BLK, D), lambda g, be: (g, 0)),
            pl.BlockSpec((1, H, D), lambda g, be: (be[g], 0, 0)),
            pl.BlockSpec((1, H, D), lambda g, be: (be[g], 0, 0)),
            pl.BlockSpec((1, D, H), lambda g, be: (be[g], 0, 0)),
        ],
        out_specs=pl.BlockSpec((BLK, D), lambda g, be: (g, 0)),
    )
    return pl.pallas_call(
        _gmm_body,
        grid_spec=grid_spec,
        out_shape=jax.ShapeDtypeStruct((PADN, D), jnp.float32),
        compiler_params=pltpu.CompilerParams(vmem_limit_bytes=100 * 2**20),
    )(bexp, xs, w1, w3, w2)


def _shared_gate_body(x_ref, gw_ref, w1_ref, w3_ref, w2_ref,
                      o_ref, tw_ref, ti_ref):
    o_ref[...] = _swiglu_block(x_ref[...], w1_ref[...], w3_ref[...], w2_ref[...])
    _gate_body(x_ref, gw_ref, tw_ref, ti_ref)


def _shared_gate(xf, gate_w, sw1, sw3, sw2):
    tb = 256
    return pl.pallas_call(
        _shared_gate_body,
        grid=(T // tb,),
        in_specs=[
            pl.BlockSpec((tb, D), lambda g: (g, 0)),
            pl.BlockSpec((E, D), lambda g: (0, 0)),
            pl.BlockSpec((SH, D), lambda g: (0, 0)),
            pl.BlockSpec((SH, D), lambda g: (0, 0)),
            pl.BlockSpec((D, SH), lambda g: (0, 0)),
        ],
        out_specs=[
            pl.BlockSpec((tb, D), lambda g: (g, 0)),
            pl.BlockSpec((tb, K), lambda g: (g, 0)),
            pl.BlockSpec((tb, K), lambda g: (g, 0)),
        ],
        out_shape=[
            jax.ShapeDtypeStruct((T, D), jnp.float32),
            jax.ShapeDtypeStruct((T, K), jnp.float32),
            jax.ShapeDtypeStruct((T, K), jnp.int32),
        ],
    )(xf, gate_w, sw1, sw3, sw2)


_CCH = 4
_CTPC = TOK_W // _CCH
_CROWS = 2 * _CTPC


def _combine_body(opad_hbm, pos_hbm, w_hbm, sh_hbm, y_hbm,
                  pos2_v, w_v, rows_a, rows_b, acc_a, acc_b,
                  gsem_a, gsem_b, osem_a, osem_b):
    wid = lax.axis_index("s") * NC + lax.axis_index("c")
    for c in range(_CCH):
        pltpu.sync_copy(pos_hbm.at[pl.ds(wid * SLOTS_W + c * _CROWS, _CROWS)],
                        pos2_v.at[c])
    pltpu.sync_copy(w_hbm.at[pl.ds(wid * SLOTS_W, SLOTS_W)], w_v)
    lane = lax.iota(jnp.int32, L)
    rows = (rows_a, rows_b)
    accs = (acc_a, acc_b)
    gsems = (gsem_a, gsem_b)
    osems = (osem_a, osem_b)

    def issue(c):
        sel = c % 2
        return pltpu.async_copy(opad_hbm.at[pos2_v.at[c]], rows[sel], gsems[sel])

    gdesc = {}
    odesc = {}
    for c in range(2):
        gdesc[c] = issue(c)
    for c in range(_CCH):
        sel = c % 2
        gdesc[c].wait()
        if c >= 2:
            odesc[c - 2].wait()
        t0 = wid * TOK_W + c * _CTPC
        rv, av = rows[sel], accs[sel]
        pltpu.sync_copy(sh_hbm.at[pl.ds(t0, _CTPC)], av)
        wlo = w_v[pl.ds(c * _CROWS, L)]
        whi = w_v[pl.ds(c * _CROWS + L, L)]
        for t in range(_CTPC):
            wc = wlo if t < L // 2 else whi
            j = (2 * t) % L
            w0 = jnp.sum(jnp.where(lane == j, wc, 0.0))
            w1v = jnp.sum(jnp.where(lane == j + 1, wc, 0.0))

            def sbody(s, _, t=t, w0=w0, w1v=w1v, rv=rv, av=av):
                for u in range(4):
                    sl = pl.ds(pl.multiple_of(s * 4 * L + u * L, L), L)
                    av[t, sl] = av[t, sl] + rv[2 * t, sl] * w0 + rv[2 * t + 1, sl] * w1v
                return 0
            lax.fori_loop(0, D // (4 * L), sbody, 0)
        odesc[c] = pltpu.async_copy(av, y_hbm.at[pl.ds(t0, _CTPC)], osems[sel])
        if c + 2 < _CCH:
            gdesc[c + 2] = issue(c + 2)
    for c in range(_CCH - 2, _CCH):
        odesc[c].wait()


def _combine(out_pad, pos, wflat, shared):
    fn = pl.kernel(
        _combine_body,
        out_type=jax.ShapeDtypeStruct((T, D), jnp.float32),
        mesh=_sc_mesh(),
        compiler_params=pltpu.CompilerParams(needs_layout_passes=False),
        scratch_types=[
            pltpu.VMEM((_CCH, _CROWS), jnp.int32),
            pltpu.VMEM((SLOTS_W,), jnp.float32),
            pltpu.VMEM((_CROWS, D), jnp.float32),
            pltpu.VMEM((_CROWS, D), jnp.float32),
            pltpu.VMEM((_CTPC, D), jnp.float32),
            pltpu.VMEM((_CTPC, D), jnp.float32),
            pltpu.SemaphoreType.DMA,
            pltpu.SemaphoreType.DMA,
            pltpu.SemaphoreType.DMA,
            pltpu.SemaphoreType.DMA,
        ],
    )
    return fn(out_pad, pos, wflat, shared)


_DBG_ROUTE_JNP = False
_DBG_COMBINE_JNP = False


def _route_jnp(eidx, xf):
    oh = (eidx[:, None] == jnp.arange(E)[None, :]).astype(jnp.int32)
    tot = oh.sum(axis=0)
    padded = ((tot + (BLK - 1)) // BLK) * BLK
    starts = jnp.cumsum(padded) - padded
    rank = jnp.take_along_axis(jnp.cumsum(oh, axis=0) - oh, eidx[:, None], axis=1)[:, 0]
    pos = starts[eidx] + rank
    lastp = jnp.max(jnp.where(tot > 0, jnp.arange(E), 0))
    b256 = jnp.arange(BEXP_N) * BLK
    bexp = jnp.full((BEXP_N,), lastp, jnp.int32)
    for e in range(E):
        bexp = jnp.where((b256 >= starts[e]) & (b256 < starts[e] + padded[e]), e, bexp)
    xs = jnp.zeros((PADN, D), jnp.float32).at[pos].set(jnp.repeat(xf, K, axis=0))
    return pos.astype(jnp.int32), bexp.astype(jnp.int32), xs


def _combine_jnp(out_pad, pos, wflat, shared):
    return (out_pad[pos[0::2]] * wflat[0::2, None]
            + out_pad[pos[1::2]] * wflat[1::2, None] + shared)


def kernel(x, gate_w, w1, w3, w2, sw1, sw3, sw2):
    orig_shape = x.shape
    xf = x.reshape(-1, D)
    shared, topk_w, topk_idx = _shared_gate(xf, gate_w, sw1, sw3, sw2)
    eidx = topk_idx.reshape(-1)
    if _DBG_ROUTE_JNP:
        pos, bexp, xs = _route_jnp(eidx, xf)
    else:
        pos, bexp, xs = _route_dispatch(eidx, xf)
    out_pad = _group_mm(bexp, xs, w1, w3, w2)
    if _DBG_COMBINE_JNP:
        y = _combine_jnp(out_pad, pos, topk_w.reshape(-1), shared)
    else:
        y = _combine(out_pad, pos, topk_w.reshape(-1), shared)
    return y.reshape(orig_shape)


# speedup vs baseline: 1.0471x; 1.0471x over previous
import functools

import jax
import jax.numpy as jnp
from jax import lax
from jax.experimental import pallas as pl
from jax.experimental.pallas import tpu as pltpu
from jax.experimental.pallas import tpu_sc as plsc

T = 2048
D = 1024
H = 2048
SH = 1024
E = 8
K = 2
N = T * K
BLK = 256
BLK_SH = 8
GMAX = 23
PADN = GMAX * BLK
BEXP_N = 32
NC, NS, L = 2, 16, 16
NW = NC * NS
SLOTS_W = N // NW
CH_W = SLOTS_W // L
TOK_W = T // NW


def _sc_mesh():
    return plsc.VectorSubcoreMesh(
        core_axis_name="c", subcore_axis_name="s", num_cores=NC, num_subcores=NS
    )


def _gate_body(x_ref, gw_ref, w_ref, i_ref):
    xb = x_ref[...]
    gw = gw_ref[...]
    logits = lax.dot_general(xb, gw, (((1,), (1,)), ((), ())),
                             preferred_element_type=jnp.float32)
    m = jnp.max(logits, axis=-1, keepdims=True)
    p = jnp.exp(logits - m)
    probs = p / jnp.sum(p, axis=-1, keepdims=True)
    lane = lax.broadcasted_iota(jnp.int32, probs.shape, 1)
    w1 = jnp.max(probs, axis=-1, keepdims=True)
    i1 = jnp.min(jnp.where(probs == w1, lane, E), axis=-1, keepdims=True)
    probs2 = jnp.where(lane == i1, -jnp.inf, probs)
    w2 = jnp.max(probs2, axis=-1, keepdims=True)
    i2 = jnp.min(jnp.where(probs2 == w2, lane, E), axis=-1, keepdims=True)
    w_ref[...] = jnp.concatenate([w1, w2], axis=1)
    i_ref[...] = jnp.concatenate([i1, i2], axis=1)


def _gate(xf, gate_w):
    tb = 256
    return pl.pallas_call(
        _gate_body,
        grid=(T // tb,),
        in_specs=[
            pl.BlockSpec((tb, D), lambda g: (g, 0)),
            pl.BlockSpec((E, D), lambda g: (0, 0)),
        ],
        out_specs=[
            pl.BlockSpec((tb, K), lambda g: (g, 0)),
            pl.BlockSpec((tb, K), lambda g: (g, 0)),
        ],
        out_shape=[
            jax.ShapeDtypeStruct((T, K), jnp.float32),
            jax.ShapeDtypeStruct((T, K), jnp.int32),
        ],
    )(xf, gate_w)


def _route_body(eidx_hbm, x_hbm, pos_hbm, bexp_hbm, xs_hbm,
                idx_v, pos_v, base_v, bexp_v, xrow_a, xrow_b,
                gsem_a, gsem_b, ssem_a, ssem_b):
    wid = lax.axis_index("s") * NC + lax.axis_index("c")
    lane = lax.iota(jnp.int32, L)
    pltpu.sync_copy(eidx_hbm, idx_v)

    def hist_upto(nchunks):
        def body(i, accs):
            v = idx_v[pl.ds(pl.multiple_of(i * L, L), L)]
            return tuple(accs[e] + (v == e).astype(jnp.int32) for e in range(E))
        accs = lax.fori_loop(0, nchunks,
                             body, tuple(jnp.zeros((L,), jnp.int32) for _ in range(E)))
        h = jnp.zeros((L,), jnp.int32)
        for e in range(E):
            h = jnp.where(lane == e, jnp.sum(accs[e]), h)
        return h

    tot = hist_upto(N // L)
    pre = hist_upto(wid * CH_W)

    padded = ((tot + (BLK - 1)) >> BLK_SH) << BLK_SH
    starts = plsc.cumsum(padded) - padded
    base_v[...] = starts + pre

    cvec = jnp.zeros((L,), jnp.int32)
    for c in range(CH_W):
        v = idx_v[pl.ds(pl.multiple_of((wid * CH_W + c) * L, L), L)]
        r = jnp.zeros((L,), jnp.int32)
        for e in range(E):
            m = v == e
            mi = m.astype(jnp.int32)
            cs = plsc.cumsum(mi)
            ce = jnp.sum(jnp.where(lane == e, cvec, 0))
            r = jnp.where(m, cs - 1 + ce, r)
            cvec = jnp.where(lane == e, cvec + jnp.sum(mi), cvec)
        g = plsc.load_gather(base_v, [v])
        pos_v[pl.ds(c * L, L)] = g + r
    pltpu.sync_copy(pos_v, pos_hbm.at[pl.ds(wid * SLOTS_W, SLOTS_W)])

    @pl.when(wid == 0)
    def _():
        lastp = jnp.max(jnp.where(tot > 0, lane, 0))
        for gi in range(BEXP_N // L):
            b256 = (lane + gi * L) * BLK
            acc = jnp.zeros((L,), jnp.int32) + lastp
            for e in range(E):
                s_e = jnp.sum(jnp.where(lane == e, starts, 0))
                p_e = jnp.sum(jnp.where(lane == e, padded, 0))
                acc = jnp.where((b256 >= s_e) & (b256 < s_e + p_e), e, acc)
            bexp_v[pl.ds(gi * L, L)] = acc
        pltpu.sync_copy(bexp_v, bexp_hbm)

    bufs = (xrow_a, xrow_b)
    gsems = (gsem_a, gsem_b)
    ssems = (ssem_a, ssem_b)

    def tokv(c):
        return (wid * SLOTS_W + c * L + lane) >> 1

    gdesc = {}
    sdesc = {}
    for c in range(min(2, CH_W)):
        gdesc[c] = pltpu.async_copy(x_hbm.at[tokv(c)], bufs[c % 2], gsems[c % 2])
    for c in range(CH_W):
        sel = c % 2
        gdesc[c].wait()
        p = pos_v[pl.ds(c * L, L)]
        sdesc[c] = pltpu.async_copy(bufs[sel], xs_hbm.at[p], ssems[sel])
        if c + 2 < CH_W:
            sdesc[c].wait()
            gdesc[c + 2] = pltpu.async_copy(x_hbm.at[tokv(c + 2)], bufs[sel], gsems[sel])
    sdesc[CH_W - 2].wait()
    sdesc[CH_W - 1].wait()


def _route_dispatch(eidx, xf):
    fn = pl.kernel(
        _route_body,
        out_type=(
            jax.ShapeDtypeStruct((N,), jnp.int32),
            jax.ShapeDtypeStruct((BEXP_N,), jnp.int32),
            jax.ShapeDtypeStruct((PADN, D), jnp.float32),
        ),
        mesh=_sc_mesh(),
        compiler_params=pltpu.CompilerParams(needs_layout_passes=False),
        scratch_types=[
            pltpu.VMEM((N,), jnp.int32),
            pltpu.VMEM((SLOTS_W,), jnp.int32),
            pltpu.VMEM((L,), jnp.int32),
            pltpu.VMEM((BEXP_N,), jnp.int32),
            pltpu.VMEM((L, D), jnp.float32),
            pltpu.VMEM((L, D), jnp.float32),
            pltpu.SemaphoreType.DMA,
            pltpu.SemaphoreType.DMA,
            pltpu.SemaphoreType.DMA,
            pltpu.SemaphoreType.DMA,
        ],
    )
    return fn(eidx, xf)


def _swiglu_block(xb, w1b, w3b, w2b):
    u = lax.dot_general(xb, w1b, (((1,), (1,)), ((), ())),
                        preferred_element_type=jnp.float32)
    v = lax.dot_general(xb, w3b, (((1,), (1,)), ((), ())),
                        preferred_element_type=jnp.float32)
    h = u * jax.nn.sigmoid(u) * v
    return lax.dot_general(h, w2b, (((1,), (1,)), ((), ())),
                           preferred_element_type=jnp.float32)


def _gmm_body(be_ref, xs_ref, w1_ref, w3_ref, w2_ref, o_ref):
    o_ref[...] = _swiglu_block(xs_ref[...], w1_ref[0], w3_ref[0], w2_ref[0])


def _group_mm(bexp, xs, w1, w3, w2):
    grid_spec = pltpu.PrefetchScalarGridSpec(
        num_scalar_prefetch=1,
        grid=(GMAX,),
        in_specs=[
            pl.BlockSpec((BLK, D), lambda g, be: (g, 0)),
            pl.BlockSpec((1, H, D), lambda g, be: (be[g], 0, 0)),
            pl.BlockSpec((1, H, D), lambda g, be: (be[g], 0, 0)),
            pl.BlockSpec((1, D, H), lambda g, be: (be[g], 0, 0)),
        ],
        out_specs=pl.BlockSpec((BLK, D), lambda g, be: (g, 0)),
    )
    return pl.pallas_call(
        _gmm_body,
        grid_spec=grid_spec,
        out_shape=jax.ShapeDtypeStruct((PADN, D), jnp.float32),
        compiler_params=pltpu.CompilerParams(vmem_limit_bytes=100 * 2**20),
    )(bexp, xs, w1, w3, w2)


def _shared_gate_body(x_ref, gw_ref, w1_ref, w3_ref, w2_ref,
                      o_ref, tw_ref, ti_ref):
    o_ref[...] = _swiglu_block(x_ref[...], w1_ref[...], w3_ref[...], w2_ref[...])
    _gate_body(x_ref, gw_ref, tw_ref, ti_ref)


def _shared_gate(xf, gate_w, sw1, sw3, sw2):
    tb = 256
    return pl.pallas_call(
        _shared_gate_body,
        grid=(T // tb,),
        in_specs=[
            pl.BlockSpec((tb, D), lambda g: (g, 0)),
            pl.BlockSpec((E, D), lambda g: (0, 0)),
            pl.BlockSpec((SH, D), lambda g: (0, 0)),
            pl.BlockSpec((SH, D), lambda g: (0, 0)),
            pl.BlockSpec((D, SH), lambda g: (0, 0)),
        ],
        out_specs=[
            pl.BlockSpec((tb, D), lambda g: (g, 0)),
            pl.BlockSpec((tb, K), lambda g: (g, 0)),
            pl.BlockSpec((tb, K), lambda g: (g, 0)),
        ],
        out_shape=[
            jax.ShapeDtypeStruct((T, D), jnp.float32),
            jax.ShapeDtypeStruct((T, K), jnp.float32),
            jax.ShapeDtypeStruct((T, K), jnp.int32),
        ],
    )(xf, gate_w, sw1, sw3, sw2)


def _combine_body(opad_hbm, pos_hbm, w_hbm, sh_hbm, y_hbm,
                  pos_v, w_v, rows_a, rows_b, sh_a, sh_b, acc_a, acc_b,
                  gsem_a, gsem_b, hsem_a, hsem_b, osem_a, osem_b):
    wid = lax.axis_index("s") * NC + lax.axis_index("c")
    pltpu.sync_copy(pos_hbm.at[pl.ds(wid * SLOTS_W, SLOTS_W)], pos_v)
    pltpu.sync_copy(w_hbm.at[pl.ds(wid * SLOTS_W, SLOTS_W)], w_v)
    lane = lax.iota(jnp.int32, L)
    rows = (rows_a, rows_b)
    shs = (sh_a, sh_b)
    accs = (acc_a, acc_b)
    gsems = (gsem_a, gsem_b)
    hsems = (hsem_a, hsem_b)
    osems = (osem_a, osem_b)
    TPC = L // 2

    def issue(c):
        sel = c % 2
        p = pos_v[pl.ds(c * L, L)]
        g = pltpu.async_copy(opad_hbm.at[p], rows[sel], gsems[sel])
        t0 = wid * TOK_W + c * TPC
        h = pltpu.async_copy(sh_hbm.at[pl.ds(t0, TPC)], shs[sel], hsems[sel])
        return g, h

    gdesc = {}
    odesc = {}
    for c in range(min(2, CH_W)):
        gdesc[c] = issue(c)
    for c in range(CH_W):
        sel = c % 2
        g, h = gdesc[c]
        g.wait()
        h.wait()
        if c >= 2:
            odesc[c - 2].wait()
        rv, sv, av = rows[sel], shs[sel], accs[sel]
        wc = w_v[pl.ds(c * L, L)]
        for t in range(TPC):
            w0 = jnp.sum(jnp.where(lane == 2 * t, wc, 0.0))
            w1v = jnp.sum(jnp.where(lane == 2 * t + 1, wc, 0.0))

            def sbody(s, _, t=t, w0=w0, w1v=w1v, rv=rv, sv=sv, av=av):
                for u in range(4):
                    sl = pl.ds(pl.multiple_of(s * 4 * L + u * L, L), L)
                    av[t, sl] = rv[2 * t, sl] * w0 + rv[2 * t + 1, sl] * w1v + sv[t, sl]
                return 0
            lax.fori_loop(0, D // (4 * L), sbody, 0)
        t0 = wid * TOK_W + c * TPC
        odesc[c] = pltpu.async_copy(av, y_hbm.at[pl.ds(t0, TPC)], osems[sel])
        if c + 2 < CH_W:
            gdesc[c + 2] = issue(c + 2)
    for c in range(max(0, CH_W - 2), CH_W):
        odesc[c].wait()


def _combine(out_pad, pos, wflat, shared):
    fn = pl.kernel(
        _combine_body,
        out_type=jax.ShapeDtypeStruct((T, D), jnp.float32),
        mesh=_sc_mesh(),
        compiler_params=pltpu.CompilerParams(needs_layout_passes=False),
        scratch_types=[
            pltpu.VMEM((SLOTS_W,), jnp.int32),
            pltpu.VMEM((SLOTS_W,), jnp.float32),
            pltpu.VMEM((L, D), jnp.float32),
            pltpu.VMEM((L, D), jnp.float32),
            pltpu.VMEM((L // 2, D), jnp.float32),
            pltpu.VMEM((L // 2, D), jnp.float32),
            pltpu.VMEM((L // 2, D), jnp.float32),
            pltpu.VMEM((L // 2, D), jnp.float32),
            pltpu.SemaphoreType.DMA,
            pltpu.SemaphoreType.DMA,
            pltpu.SemaphoreType.DMA,
            pltpu.SemaphoreType.DMA,
            pltpu.SemaphoreType.DMA,
            pltpu.SemaphoreType.DMA,
        ],
    )
    return fn(out_pad, pos, wflat, shared)


_DBG_ROUTE_JNP = False
_DBG_COMBINE_JNP = False


def _route_jnp(eidx, xf):
    oh = (eidx[:, None] == jnp.arange(E)[None, :]).astype(jnp.int32)
    tot = oh.sum(axis=0)
    padded = ((tot + (BLK - 1)) // BLK) * BLK
    starts = jnp.cumsum(padded) - padded
    rank = jnp.take_along_axis(jnp.cumsum(oh, axis=0) - oh, eidx[:, None], axis=1)[:, 0]
    pos = starts[eidx] + rank
    lastp = jnp.max(jnp.where(tot > 0, jnp.arange(E), 0))
    b256 = jnp.arange(BEXP_N) * BLK
    bexp = jnp.full((BEXP_N,), lastp, jnp.int32)
    for e in range(E):
        bexp = jnp.where((b256 >= starts[e]) & (b256 < starts[e] + padded[e]), e, bexp)
    xs = jnp.zeros((PADN, D), jnp.float32).at[pos].set(jnp.repeat(xf, K, axis=0))
    return pos.astype(jnp.int32), bexp.astype(jnp.int32), xs


def _combine_jnp(out_pad, pos, wflat, shared):
    return (out_pad[pos[0::2]] * wflat[0::2, None]
            + out_pad[pos[1::2]] * wflat[1::2, None] + shared)


def kernel(x, gate_w, w1, w3, w2, sw1, sw3, sw2):
    orig_shape = x.shape
    xf = x.reshape(-1, D)
    shared, topk_w, topk_idx = _shared_gate(xf, gate_w, sw1, sw3, sw2)
    eidx = topk_idx.reshape(-1)
    if _DBG_ROUTE_JNP:
        pos, bexp, xs = _route_jnp(eidx, xf)
    else:
        pos, bexp, xs = _route_dispatch(eidx, xf)
    out_pad = _group_mm(bexp, xs, w1, w3, w2)
    if _DBG_COMBINE_JNP:
        y = _combine_jnp(out_pad, pos, topk_w.reshape(-1), shared)
    else:
        y = _combine(out_pad, pos, topk_w.reshape(-1), shared)
    return y.reshape(orig_shape)
